# baseline (speedup 1.0000x reference)
import functools

import jax
import jax.numpy as jnp
from jax import lax
from jax.experimental import pallas as pl
from jax.experimental.pallas import tpu as pltpu
from jax.experimental.pallas import tpu_sc as plsc

SEQ = 200
EMB = 64
BPW = 128
CL = 2
SCALE = 8.0

_info = plsc.get_sparse_core_info()
_NC, _NS, _NL = _info.num_cores, _info.num_subcores, _info.num_lanes
_NW = _NC * _NS
_ND = EMB // _NL
_NB = BPW // _NL


def _build(batch: int):
  assert batch == BPW * _NW

  mesh = plsc.VectorSubcoreMesh(core_axis_name="c", subcore_axis_name="s")

  @functools.partial(
      pl.kernel,
      mesh=mesh,
      compiler_params=pltpu.CompilerParams(
          use_tc_tiling_on_sc=False, needs_layout_passes=False),
      out_type=jax.ShapeDtypeStruct(
          (SEQ, EMB // 8, batch // BPW, 8, BPW), jnp.float32),
      scratch_types=[
          pltpu.VMEM((BPW, SEQ), jnp.int32),
          pltpu.VMEM((2, CL * BPW), jnp.int32),
          pltpu.VMEM((CL * BPW, EMB), jnp.float32),
          pltpu.VMEM((CL * BPW, EMB), jnp.float32),
          pltpu.VMEM((CL, EMB // 8, 8, BPW), jnp.float32),
          pltpu.VMEM((CL, EMB // 8, 8, BPW), jnp.float32),
          pltpu.VMEM((SEQ, EMB), jnp.float32),
          pltpu.SemaphoreType.DMA,
          pltpu.SemaphoreType.DMA,
          pltpu.SemaphoreType.DMA,
      ],
  )
  def emb(idx_hbm, table_hbm, pos_hbm, out_hbm,
          slab_v, cidx_v, g0, g1, t0, t1, pos_v, sg, so0, so1):
    wid = lax.axis_index("s") * _NC + lax.axis_index("c")
    b0 = wid * BPW
    gbuf = (g0, g1)
    tbuf = (t0, t1)
    so = (so0, so1)

    pltpu.sync_copy(idx_hbm.at[pl.ds(b0, BPW)], slab_v)
    pltpu.sync_copy(pos_hbm, pos_v)

    lanes = lax.iota(jnp.int32, _NL)
    zeros = lanes * 0

    def extract_idx(l0, slot):
      for l in range(CL):
        lv = zeros + (l0 + l)
        for bb in range(_NB):
          v = plsc.load_gather(slab_v, [lanes + bb * _NL, lv])
          cidx_v[slot, pl.ds(l * BPW + bb * _NL, _NL)] = v

    def gather(slot):
      return pltpu.make_async_copy(
          table_hbm.at[cidx_v.at[slot]], gbuf[slot], sg)

    def out_copy(l0, slot):
      return pltpu.make_async_copy(
          tbuf[slot], out_hbm.at[pl.ds(l0, CL), :, wid], so[slot])

    def compute(l0, slot):
      g = gbuf[slot]
      t = tbuf[slot]

      def body(k, c):
        kv = zeros + k
        fh8 = k // 8
        fl = k % 8
        for l in range(CL):
          for d in range(_ND):
            pv = pos_v[l0 + l, pl.ds(d * _NL, _NL)]
            p = lax.gather(
                pv, kv[:, None],
                dimension_numbers=lax.GatherDimensionNumbers(
                    offset_dims=(), collapsed_slice_dims=(0,),
                    start_index_map=(0,)),
                slice_sizes=(1,),
                mode=lax.GatherScatterMode.PROMISE_IN_BOUNDS)
            fv = kv + d * _NL
            fh = 2 * d + fh8
            for bb in range(_NB):
              v = plsc.load_gather(g, [lanes + (l * BPW + bb * _NL), fv])
              t[l, fh, fl, pl.ds(bb * _NL, _NL)] = v * SCALE + p
        return c

      lax.fori_loop(0, _NL, body, 0)

    n_chunks = SEQ // CL

    def step(g_i, slot):
      l0 = g_i * CL
      gather(slot).wait()

      @pl.when(g_i + 1 < n_chunks)
      def _():
        extract_idx(l0 + CL, 1 - slot)
        gather(1 - slot).start()

      @pl.when(g_i >= 2)
      def _():
        out_copy((g_i - 2) * CL, slot).wait()

      compute(l0, slot)
      out_copy(l0, slot).start()

    extract_idx(0, 0)
    gather(0).start()

    def pair(p, c):
      g_i = 2 * p
      step(g_i, 0)
      step(g_i + 1, 1)
      return c

    lax.fori_loop(0, n_chunks // 2, pair, 0)

    out_copy((n_chunks - 2) * CL, 0).wait()
    out_copy((n_chunks - 1) * CL, 1).wait()

  return emb


def kernel(inputs, token_table, position_table):
  batch = inputs.shape[0]
  emb = _build(batch)
  out5 = emb(inputs, token_table, position_table)
  return out5.transpose(2, 4, 0, 1, 3).reshape(batch, SEQ, EMB)


# speedup vs baseline: 1.6927x; 1.6927x over previous
import functools

import jax
import jax.numpy as jnp
from jax import lax
from jax.experimental import pallas as pl
from jax.experimental.pallas import tpu as pltpu
from jax.experimental.pallas import tpu_sc as plsc

SEQ = 200
EMB = 64
SCALE = 8.0

_info = plsc.get_sparse_core_info()
_NC, _NS, _NL = _info.num_cores, _info.num_subcores, _info.num_lanes
_NW = _NC * _NS
_ND = EMB // _NL


def _build(batch: int, cseq: int):
  spw = batch // _NW
  n_chunks = spw // cseq
  assert batch % _NW == 0 and spw % cseq == 0
  assert n_chunks >= 4 and n_chunks % 2 == 0

  mesh = plsc.VectorSubcoreMesh(core_axis_name="c", subcore_axis_name="s")

  @functools.partial(
      pl.kernel,
      mesh=mesh,
      compiler_params=pltpu.CompilerParams(use_tc_tiling_on_sc=False),
      out_type=jax.ShapeDtypeStruct((batch, SEQ, EMB), jnp.float32),
      scratch_types=[
          pltpu.VMEM((2, cseq, SEQ), jnp.int32),
          pltpu.VMEM((cseq, SEQ, EMB), jnp.float32),
          pltpu.VMEM((cseq, SEQ, EMB), jnp.float32),
          pltpu.VMEM((SEQ, EMB), jnp.float32),
          pltpu.SemaphoreType.DMA,
          pltpu.SemaphoreType.DMA,
          pltpu.SemaphoreType.DMA,
      ],
  )
  def emb(idx_hbm, table_hbm, pos_hbm, out_hbm,
          idx_v, rows0_v, rows1_v, pos_v, sg, si, so):
    wid = lax.axis_index("s") * _NC + lax.axis_index("c")
    b0 = wid * spw
    pltpu.sync_copy(pos_hbm, pos_v)
    rows = (rows0_v, rows1_v)

    def idx_copy(g, slot):
      return pltpu.make_async_copy(
          idx_hbm.at[pl.ds(b0 + g * cseq, cseq)], idx_v.at[slot], si)

    def gathers(slot):
      return [
          pltpu.make_async_copy(
              table_hbm.at[idx_v.at[slot, s]], rows[slot].at[s], sg)
          for s in range(cseq)
      ]

    def out_copy(g, slot):
      return pltpu.make_async_copy(
          rows[slot], out_hbm.at[pl.ds(b0 + g * cseq, cseq)], so)

    def start_gathers(slot):
      for c in gathers(slot):
        c.start()

    def wait_gathers(slot):
      for c in gathers(slot):
        c.wait()

    def compute(buf):
      def body(l, c):
        pv = [pos_v[l, pl.ds(d * _NL, _NL)] for d in range(_ND)]
        for s in range(cseq):
          for d in range(_ND):
            sl = pl.ds(d * _NL, _NL)
            buf[s, l, sl] = buf[s, l, sl] * SCALE + pv[d]
        return c

      lax.fori_loop(0, SEQ, body, 0, unroll=2)

    def step(g, slot):
      wait_gathers(slot)
      idx_copy(g + 1, 1 - slot).wait()
      out_copy(g - 1, 1 - slot).wait()
      start_gathers(1 - slot)

      @pl.when(g + 2 < n_chunks)
      def _():
        idx_copy(g + 2, slot).start()

      compute(rows[slot])
      out_copy(g, slot).start()

    pltpu.sync_copy(idx_hbm.at[pl.ds(b0, cseq)], idx_v.at[0])
    start_gathers(0)
    idx_copy(1, 1).start()
    wait_gathers(0)
    idx_copy(1, 1).wait()
    start_gathers(1)
    idx_copy(2, 0).start()
    compute(rows[0])
    out_copy(0, 0).start()

    def pair(p, c):
      g = 1 + 2 * p
      step(g, 1)
      step(g + 1, 0)
      return c

    lax.fori_loop(0, (n_chunks - 2) // 2, pair, 0)

    wait_gathers(1)
    out_copy(n_chunks - 2, 0).wait()
    compute(rows[1])
    out_copy(n_chunks - 1, 1).start()
    out_copy(n_chunks - 1, 1).wait()

  return emb


def kernel(inputs, token_table, position_table):
  batch = inputs.shape[0]
  emb = _build(batch, cseq=4)
  return emb(inputs, token_table, position_table)


# speedup vs baseline: 2.2421x; 1.3246x over previous
import functools

import jax
import jax.numpy as jnp
from jax import lax
from jax.experimental import pallas as pl
from jax.experimental.pallas import tpu as pltpu
from jax.experimental.pallas import tpu_sc as plsc

SEQ = 200
EMB = 64
SCALE = 8.0

_info = plsc.get_sparse_core_info()
_NC, _NS, _NL = _info.num_cores, _info.num_subcores, _info.num_lanes
_NW = _NC * _NS
_ND = EMB // _NL


def _build(batch: int, cseq: int):
  spw = batch // _NW
  n_chunks = spw // cseq
  assert batch % _NW == 0 and spw % cseq == 0
  assert n_chunks >= 4 and n_chunks % 2 == 0

  mesh = plsc.VectorSubcoreMesh(core_axis_name="c", subcore_axis_name="s")

  @functools.partial(
      pl.kernel,
      mesh=mesh,
      compiler_params=pltpu.CompilerParams(use_tc_tiling_on_sc=False),
      out_type=jax.ShapeDtypeStruct((batch, SEQ, 2 * EMB), jnp.float32),
      scratch_types=[
          pltpu.VMEM((2, cseq, SEQ), jnp.int32),
          pltpu.VMEM((cseq, SEQ, EMB), jnp.float32),
          pltpu.VMEM((cseq, SEQ, EMB), jnp.float32),
          pltpu.VMEM((SEQ, EMB), jnp.float32),
          pltpu.SemaphoreType.DMA,
          pltpu.SemaphoreType.DMA,
          pltpu.SemaphoreType.DMA,
      ],
  )
  def emb(idx_hbm, table_hbm, pos_hbm, out_hbm,
          idx_v, rows0_v, rows1_v, pos_v, sg, si, so):
    wid = lax.axis_index("s") * _NC + lax.axis_index("c")
    b0 = wid * spw
    pltpu.sync_copy(pos_hbm, pos_v)
    rows = (rows0_v, rows1_v)

    def idx_copy(g, slot):
      return pltpu.make_async_copy(
          idx_hbm.at[pl.ds(b0 + g * cseq, cseq)], idx_v.at[slot], si)

    def gathers(slot):
      return [
          pltpu.make_async_copy(
              table_hbm.at[idx_v.at[slot, s]], rows[slot].at[s], sg)
          for s in range(cseq)
      ]

    def out_copy(g, slot):
      return pltpu.make_async_copy(
          rows[slot],
          out_hbm.at[pl.ds(b0 + g * cseq, cseq), :, pl.ds(0, EMB)], so)

    def start_gathers(slot):
      for c in gathers(slot):
        c.start()

    def wait_gathers(slot):
      for c in gathers(slot):
        c.wait()

    def compute(buf):
      def body(l, c):
        pv = [pos_v[l, pl.ds(d * _NL, _NL)] for d in range(_ND)]
        for s in range(cseq):
          for d in range(_ND):
            sl = pl.ds(d * _NL, _NL)
            buf[s, l, sl] = buf[s, l, sl] * SCALE + pv[d]
        return c

      lax.fori_loop(0, SEQ, body, 0, unroll=2)

    def step(g, slot):
      wait_gathers(slot)
      idx_copy(g + 1, 1 - slot).wait()
      out_copy(g - 1, 1 - slot).wait()
      start_gathers(1 - slot)

      @pl.when(g + 2 < n_chunks)
      def _():
        idx_copy(g + 2, slot).start()

      compute(rows[slot])
      out_copy(g, slot).start()

    pltpu.sync_copy(idx_hbm.at[pl.ds(b0, cseq)], idx_v.at[0])
    start_gathers(0)
    idx_copy(1, 1).start()
    wait_gathers(0)
    idx_copy(1, 1).wait()
    start_gathers(1)
    idx_copy(2, 0).start()
    compute(rows[0])
    out_copy(0, 0).start()

    def pair(p, c):
      g = 1 + 2 * p
      step(g, 1)
      step(g + 1, 0)
      return c

    lax.fori_loop(0, (n_chunks - 2) // 2, pair, 0)

    wait_gathers(1)
    out_copy(n_chunks - 2, 0).wait()
    compute(rows[1])
    out_copy(n_chunks - 1, 1).start()
    out_copy(n_chunks - 1, 1).wait()

  return emb


def kernel(inputs, token_table, position_table):
  batch = inputs.shape[0]
  emb = _build(batch, cseq=4)
  return emb(inputs, token_table, position_table)[:, :, :EMB]
